# baseline (speedup 1.0000x reference)
import functools

import jax
import jax.numpy as jnp
from jax import lax
from jax.experimental import pallas as pl
from jax.experimental.pallas import tpu as pltpu
from jax.experimental.pallas import tpu_sc as plsc

N = 10000
E = 320000
D = 128
EPS = 0.3

NC = 2
NS = 16
LANES = 16

NPAD = 10240
ROWS_PER_TILE = NPAD // NS
EPT = 10240
EPAD = NC * NS * EPT
BLK = 64
NBLK = EPT // BLK
CPB = 8
ECH = CPB * BLK
NCHUNK = EPT // ECH
NRING = 3


def _prep_body(x_ref, w1t_ref, b1_ref, g_ref, bgv_ref, h_ref, qr_ref):
    h = jnp.maximum(jnp.dot(x_ref[...], w1t_ref[...],
                            preferred_element_type=jnp.float32)
                    + b1_ref[...], 0.0)
    h_ref[...] = h
    qr_ref[...] = jnp.dot(h, g_ref[...],
                          preferred_element_type=jnp.float32) + bgv_ref[...]


def _mid_body(h_ref, agg_ref, g_ref, bgv_ref, h1_ref, qr_ref):
    h1 = EPS * h_ref[...] + agg_ref[0] + agg_ref[1]
    h1_ref[...] = h1
    qr_ref[...] = jnp.dot(h1, g_ref[...],
                          preferred_element_type=jnp.float32) + bgv_ref[...]


def _final_body(h_ref, agg_ref, w2t_ref, b2_ref, out_ref):
    h2 = EPS * h_ref[...] + agg_ref[0] + agg_ref[1]
    o = jnp.dot(h2, w2t_ref[...], preferred_element_type=jnp.float32) \
        + b2_ref[...]
    m = jnp.max(o, axis=1, keepdims=True)
    ls = jnp.log(jnp.sum(jnp.exp(o - m), axis=1, keepdims=True))
    out_ref[...] = o - m - ls


def _deg_body(dst4_hbm, out_hbm, zero_v, ones_v, idx_v, deg_sp):
    c = lax.axis_index("c")
    s = lax.axis_index("s")
    w = c * NS + s

    def _z(i, _):
        zero_v[pl.ds(i * LANES, LANES)] = jnp.zeros((LANES,), jnp.float32)
        return 0
    lax.fori_loop(0, ROWS_PER_TILE // LANES, _z, 0)
    for i in range(BLK // LANES):
        ones_v[pl.ds(i * LANES, LANES)] = jnp.ones((LANES,), jnp.float32)

    pltpu.sync_copy(zero_v, deg_sp.at[pl.ds(s * ROWS_PER_TILE,
                                            ROWS_PER_TILE)])
    pltpu.sync_copy(dst4_hbm.at[w], idx_v)
    plsc.subcore_barrier()

    def _blk(j, _):
        pltpu.sync_copy(ones_v, deg_sp.at[idx_v.at[j]], add=True)
        return 0
    lax.fori_loop(0, NBLK, _blk, 0)

    plsc.subcore_barrier()
    pltpu.sync_copy(deg_sp.at[pl.ds(s * ROWS_PER_TILE, ROWS_PER_TILE)],
                    out_hbm.at[c, pl.ds(s * ROWS_PER_TILE, ROWS_PER_TILE)])


def _gate_body(qd_hbm, rs_hbm, deg_hbm, srcf_hbm, dstf_hbm, e_hbm,
               qd_v, rs_v, nrm_v, sidx_v, didx_v, e_v, sem):
    c = lax.axis_index("c")
    s = lax.axis_index("s")
    w = c * NS + s

    pltpu.async_copy(qd_hbm, qd_v, sem)
    pltpu.async_copy(rs_hbm, rs_v, sem)
    pltpu.async_copy(deg_hbm.at[0], nrm_v, sem)
    pltpu.async_copy(deg_hbm.at[1], e_v.at[pl.ds(0, NPAD)], sem)
    pltpu.async_copy(srcf_hbm.at[w], sidx_v, sem)
    pltpu.async_copy(dstf_hbm.at[w], didx_v, sem)
    pltpu.make_async_copy(qd_hbm, qd_v, sem).wait()
    pltpu.make_async_copy(rs_hbm, rs_v, sem).wait()
    pltpu.make_async_copy(deg_hbm.at[0], nrm_v, sem).wait()
    pltpu.make_async_copy(deg_hbm.at[1], e_v.at[pl.ds(0, NPAD)], sem).wait()
    pltpu.make_async_copy(srcf_hbm.at[w], sidx_v, sem).wait()
    pltpu.make_async_copy(dstf_hbm.at[w], didx_v, sem).wait()

    def _nrm(g, _):
        sl = pl.ds(g * LANES, LANES)
        x = jnp.maximum(nrm_v[sl] + e_v[sl], 1.0)
        yi = 0x5F3759DF - (plsc.bitcast(x, jnp.int32) >> 1)
        y = plsc.bitcast(yi, jnp.float32)
        for _ in range(3):
            y = y * (1.5 - 0.5 * x * y * y)
        nrm_v[sl] = y
        return 0
    lax.fori_loop(0, NPAD // LANES, _nrm, 0)

    def _grp(g, _):
        sl = pl.ds(g * LANES, LANES)
        sidx = sidx_v[sl]
        didx = didx_v[sl]
        al = (plsc.load_gather(qd_v, [didx])
              + plsc.load_gather(rs_v, [sidx]))
        t = 1.0 - 2.0 / (jnp.exp(2.0 * al) + 1.0)
        e_v[sl] = (t * plsc.load_gather(nrm_v, [sidx])
                   * plsc.load_gather(nrm_v, [didx]))
        return 0
    lax.fori_loop(0, EPT // LANES, _grp, 0)
    pltpu.sync_copy(e_v, e_hbm.at[w])


def _fa_body(h_hbm, src4_hbm, dst4_hbm, e4_hbm, out_hbm,
             src_v, dst_v, e_v, rows_v, sem_g, sem_s, sem_c, agg_sp):
    c = lax.axis_index("c")
    s = lax.axis_index("s")
    w = c * NS + s

    def _z(i, _):
        for k in range(D // LANES):
            rows_v[0, i, pl.ds(k * LANES, LANES)] = (
                jnp.zeros((LANES,), jnp.float32))
        return 0
    lax.fori_loop(0, BLK, _z, 0)
    for r in range(ROWS_PER_TILE // BLK):
        pltpu.sync_copy(rows_v.at[0],
                        agg_sp.at[pl.ds(s * ROWS_PER_TILE + r * BLK, BLK)])

    pltpu.sync_copy(src4_hbm.at[w], src_v)
    pltpu.sync_copy(dst4_hbm.at[w], dst_v)
    pltpu.async_copy(e4_hbm.at[w, 0], e_v.at[0], sem_c.at[0])
    plsc.subcore_barrier()

    def _iter(j, _):
        b = j % NRING
        jl = j % CPB
        cc = j // CPB
        cp = cc % 2

        @pl.when(jnp.logical_and(j < NBLK, jl == 0))
        def _stage():
            pltpu.make_async_copy(e4_hbm.at[w, cc], e_v.at[cp],
                                  sem_c.at[cp]).wait()

            @pl.when(cc + 1 < NCHUNK)
            def _next():
                pltpu.async_copy(e4_hbm.at[w, cc + 1], e_v.at[1 - cp],
                                 sem_c.at[1 - cp])

            @pl.when(j == 0)
            def _first():
                pltpu.async_copy(
                    h_hbm.at[src_v.at[pl.ds(0, BLK)]], rows_v.at[0],
                    sem_g.at[0])

        @pl.when(j >= 2)
        def _drain():
            b2 = (j - 2) % NRING
            for q in range(BLK // LANES):
                ivec = dst_v[pl.ds((j - 2) * BLK + q * LANES, LANES)]
                pltpu.make_async_copy(
                    rows_v.at[b2, pl.ds(q * LANES, LANES)],
                    agg_sp.at[ivec], sem_s.at[b2]).wait()

        @pl.when(j < NBLK)
        def _work():
            pltpu.make_async_copy(
                h_hbm.at[src_v.at[pl.ds(j * BLK, BLK)]], rows_v.at[b],
                sem_g.at[b]).wait()

            @pl.when(j + 1 < NBLK)
            def _pref():
                b1 = (j + 1) % NRING
                pltpu.async_copy(
                    h_hbm.at[src_v.at[pl.ds((j + 1) * BLK, BLK)]],
                    rows_v.at[b1], sem_g.at[b1])

            base = jl * BLK
            for g in range(BLK // LANES):
                e16 = e_v[cp, pl.ds(base + g * LANES, LANES)]
                for u in range(LANES):
                    ii = g * LANES + u
                    ee = e16[u]
                    for k in range(D // LANES):
                        fl = pl.ds(k * LANES, LANES)
                        rows_v[b, ii, fl] = rows_v[b, ii, fl] * ee

            for q in range(BLK // LANES):
                ivec = dst_v[pl.ds(j * BLK + q * LANES, LANES)]
                pltpu.async_copy(rows_v.at[b, pl.ds(q * LANES, LANES)],
                                 agg_sp.at[ivec], sem_s.at[b], add=True)
        return 0
    lax.fori_loop(0, NBLK + 2, _iter, 0)

    plsc.subcore_barrier()
    pltpu.sync_copy(agg_sp.at[pl.ds(s * ROWS_PER_TILE, ROWS_PER_TILE)],
                    out_hbm.at[c, pl.ds(s * ROWS_PER_TILE, ROWS_PER_TILE)])


_SC_MESH = plsc.VectorSubcoreMesh(core_axis_name="c", subcore_axis_name="s",
                                  num_cores=NC, num_subcores=NS)
_SC_PARAMS = pltpu.CompilerParams(needs_layout_passes=False)

_deg_kernel = pl.kernel(
    _deg_body,
    out_type=jax.ShapeDtypeStruct((NC, NPAD), jnp.float32),
    mesh=_SC_MESH,
    compiler_params=_SC_PARAMS,
    scratch_types=[
        pltpu.VMEM((ROWS_PER_TILE,), jnp.float32),
        pltpu.VMEM((BLK,), jnp.float32),
        pltpu.VMEM((NBLK, BLK), jnp.int32),
        pltpu.VMEM_SHARED((NPAD,), jnp.float32),
    ],
)

_gate_kernel = pl.kernel(
    _gate_body,
    out_type=jax.ShapeDtypeStruct((NC * NS, EPT), jnp.float32),
    mesh=_SC_MESH,
    compiler_params=_SC_PARAMS,
    scratch_types=[
        pltpu.VMEM((NPAD,), jnp.float32),
        pltpu.VMEM((NPAD,), jnp.float32),
        pltpu.VMEM((NPAD,), jnp.float32),
        pltpu.VMEM((EPT,), jnp.int32),
        pltpu.VMEM((EPT,), jnp.int32),
        pltpu.VMEM((EPT,), jnp.float32),
        pltpu.SemaphoreType.DMA,
    ],
)

_fa_kernel = pl.kernel(
    _fa_body,
    out_type=jax.ShapeDtypeStruct((NC, NPAD, D), jnp.float32),
    mesh=_SC_MESH,
    compiler_params=_SC_PARAMS,
    scratch_types=[
        pltpu.VMEM((EPT,), jnp.int32),
        pltpu.VMEM((EPT,), jnp.int32),
        pltpu.VMEM((2, ECH), jnp.float32),
        pltpu.VMEM((NRING, BLK, D), jnp.float32),
        pltpu.SemaphoreType.DMA((NRING,)),
        pltpu.SemaphoreType.DMA((NRING,)),
        pltpu.SemaphoreType.DMA((2,)),
        pltpu.VMEM_SHARED((NPAD, D), jnp.float32),
    ],
)


def _tc_call(body, grid, in_specs, out_specs, out_shape):
    return pl.pallas_call(body, grid=grid, in_specs=in_specs,
                          out_specs=out_specs, out_shape=out_shape)


def kernel(x, edge_index, W1, b1, Wg0, bg0, Wg1, bg1, W2, b2):
    src = edge_index[0]
    dst = edge_index[1]
    pad_e = EPAD - E
    pad_idx = N + (jnp.arange(pad_e, dtype=jnp.int32) % (NPAD - N))
    src3 = jnp.concatenate([src, pad_idx]).reshape(NC * NS, NBLK, BLK)
    dst3 = jnp.concatenate([dst, pad_idx]).reshape(NC * NS, NBLK, BLK)
    srcf = src3.reshape(NC * NS, NCHUNK, ECH)
    dstf = dst3.reshape(NC * NS, NCHUNK, ECH)
    srct = src3.reshape(NC * NS, EPT)
    dstt = dst3.reshape(NC * NS, EPT)
    x_pad = jnp.zeros((NPAD, D), jnp.float32).at[:N].set(x)

    w1t = W1.T
    w2t = W2.T
    g0 = jnp.zeros((D, D), jnp.float32)
    g0 = g0.at[:, 0].set(Wg0[0, :D]).at[:, 1].set(Wg0[0, D:])
    g1 = jnp.zeros((D, D), jnp.float32)
    g1 = g1.at[:, 0].set(Wg1[0, :D]).at[:, 1].set(Wg1[0, D:])
    bgv0 = jnp.zeros((1, D), jnp.float32).at[0, 0].set(bg0[0])
    bgv1 = jnp.zeros((1, D), jnp.float32).at[0, 0].set(bg1[0])
    b1r = b1.reshape(1, D)
    b2r = b2.reshape(1, D)

    deg2 = _deg_kernel(dst3)

    RB = 2048
    GP = NPAD // RB
    full = lambda shape: pl.BlockSpec(shape, lambda i: (0,) * len(shape))
    rows = pl.BlockSpec((RB, D), lambda i: (i, 0))

    h_pad, qr0 = _tc_call(
        _prep_body, (GP,),
        [rows, full((D, D)), full((1, D)), full((D, D)), full((1, D))],
        [rows, rows],
        [jax.ShapeDtypeStruct((NPAD, D), jnp.float32),
         jax.ShapeDtypeStruct((NPAD, D), jnp.float32)],
    )(x_pad, w1t, b1r, g0, bgv0)

    e0 = _gate_kernel(qr0[:, 0], qr0[:, 1], deg2, srct, dstt)
    agg0 = _fa_kernel(h_pad, srct, dstt, e0.reshape(NC * NS, NCHUNK, ECH))

    h1_pad, qr1 = _tc_call(
        _mid_body, (GP,),
        [rows, pl.BlockSpec((NC, RB, D), lambda i: (0, i, 0)),
         full((D, D)), full((1, D))],
        [rows, rows],
        [jax.ShapeDtypeStruct((NPAD, D), jnp.float32),
         jax.ShapeDtypeStruct((NPAD, D), jnp.float32)],
    )(h_pad, agg0, g1, bgv1)

    e1 = _gate_kernel(qr1[:, 0], qr1[:, 1], deg2, srct, dstt)
    agg1 = _fa_kernel(h1_pad, srct, dstt, e1.reshape(NC * NS, NCHUNK, ECH))

    RB2 = 2000
    out = _tc_call(
        _final_body, (N // RB2,),
        [pl.BlockSpec((RB2, D), lambda i: (i, 0)),
         pl.BlockSpec((NC, RB2, D), lambda i: (0, i, 0)),
         full((D, D)), full((1, D))],
        pl.BlockSpec((RB2, D), lambda i: (i, 0)),
        jax.ShapeDtypeStruct((N, D), jnp.float32),
    )(h_pad, agg1, w2t, b2r)

    return out


# speedup vs baseline: 1.3659x; 1.3659x over previous
import functools

import jax
import jax.numpy as jnp
from jax import lax
from jax.experimental import pallas as pl
from jax.experimental.pallas import tpu as pltpu
from jax.experimental.pallas import tpu_sc as plsc

N = 10000
E = 320000
D = 128
EPS = 0.3

NC = 2
NS = 16
LANES = 16

NPAD = 10240
ROWS_PER_TILE = NPAD // NS
EPT = 10240
EPAD = NC * NS * EPT
BLK = 64
NBLK = EPT // BLK
CPB = 8
ECH = CPB * BLK
NCHUNK = EPT // ECH
NRING = 4


def _prep_body(x_ref, w1t_ref, b1_ref, g_ref, bgv_ref, deg_ref,
               h_ref, qr_ref, nrm_ref):
    h = jnp.maximum(jnp.dot(x_ref[...], w1t_ref[...],
                            preferred_element_type=jnp.float32)
                    + b1_ref[...], 0.0)
    h_ref[...] = h
    qr_ref[...] = jnp.dot(h, g_ref[...],
                          preferred_element_type=jnp.float32) + bgv_ref[...]
    deg = deg_ref[0] + deg_ref[1]
    nrm_ref[...] = 1.0 / jnp.sqrt(jnp.maximum(deg, 1.0))


def _mid_body(h_ref, agg_ref, g_ref, bgv_ref, h1_ref, qr_ref):
    h1 = EPS * h_ref[...] + agg_ref[0] + agg_ref[1]
    h1_ref[...] = h1
    qr_ref[...] = jnp.dot(h1, g_ref[...],
                          preferred_element_type=jnp.float32) + bgv_ref[...]


def _final_body(h_ref, agg_ref, w2t_ref, b2_ref, out_ref):
    h2 = EPS * h_ref[...] + agg_ref[0] + agg_ref[1]
    o = jnp.dot(h2, w2t_ref[...], preferred_element_type=jnp.float32) \
        + b2_ref[...]
    m = jnp.max(o, axis=1, keepdims=True)
    ls = jnp.log(jnp.sum(jnp.exp(o - m), axis=1, keepdims=True))
    out_ref[...] = o - m - ls


def _deg_body(dst4_hbm, out_hbm, zero_v, ones_v, idx_v, deg_sp):
    c = lax.axis_index("c")
    s = lax.axis_index("s")
    w = c * NS + s

    def _z(i, _):
        zero_v[pl.ds(i * LANES, LANES)] = jnp.zeros((LANES,), jnp.float32)
        return 0
    lax.fori_loop(0, ROWS_PER_TILE // LANES, _z, 0)
    for i in range(BLK // LANES):
        ones_v[pl.ds(i * LANES, LANES)] = jnp.ones((LANES,), jnp.float32)

    pltpu.sync_copy(zero_v, deg_sp.at[pl.ds(s * ROWS_PER_TILE,
                                            ROWS_PER_TILE)])
    pltpu.sync_copy(dst4_hbm.at[w], idx_v)
    plsc.subcore_barrier()

    def _blk(j, _):
        pltpu.sync_copy(ones_v, deg_sp.at[idx_v.at[j]], add=True)
        return 0
    lax.fori_loop(0, NBLK, _blk, 0)

    plsc.subcore_barrier()
    pltpu.sync_copy(deg_sp.at[pl.ds(s * ROWS_PER_TILE, ROWS_PER_TILE)],
                    out_hbm.at[c, pl.ds(s * ROWS_PER_TILE, ROWS_PER_TILE)])


def _gate_body(qd_hbm, rs_hbm, nrm_hbm, srcf_hbm, dstf_hbm, e_hbm,
               qd_v, rs_v, nrm_v, sidx_v, didx_v, e_v, sem):
    c = lax.axis_index("c")
    s = lax.axis_index("s")
    w = c * NS + s

    pltpu.async_copy(qd_hbm, qd_v, sem)
    pltpu.async_copy(rs_hbm, rs_v, sem)
    pltpu.async_copy(nrm_hbm, nrm_v, sem)
    pltpu.async_copy(srcf_hbm.at[w], sidx_v, sem)
    pltpu.async_copy(dstf_hbm.at[w], didx_v, sem)
    pltpu.make_async_copy(qd_hbm, qd_v, sem).wait()
    pltpu.make_async_copy(rs_hbm, rs_v, sem).wait()
    pltpu.make_async_copy(nrm_hbm, nrm_v, sem).wait()
    pltpu.make_async_copy(srcf_hbm.at[w], sidx_v, sem).wait()
    pltpu.make_async_copy(dstf_hbm.at[w], didx_v, sem).wait()

    def _grp(g, _):
        sl = pl.ds(g * LANES, LANES)
        sidx = sidx_v[sl]
        didx = didx_v[sl]
        al = (plsc.load_gather(qd_v, [didx])
              + plsc.load_gather(rs_v, [sidx]))
        t = 1.0 - 2.0 / (jnp.exp(2.0 * al) + 1.0)
        e_v[sl] = (t * plsc.load_gather(nrm_v, [sidx])
                   * plsc.load_gather(nrm_v, [didx]))
        return 0
    lax.fori_loop(0, EPT // LANES, _grp, 0)
    pltpu.sync_copy(e_v, e_hbm.at[w])


def _fa_body(h_hbm, pk_hbm, e4_hbm, out_hbm,
             pk_v, sidx_v, e_v, rows_v, sem_g, sem_s, sem_c, agg_sp):
    c = lax.axis_index("c")
    s = lax.axis_index("s")
    w = c * NS + s

    def _z(i, _):
        for k in range(D // LANES):
            rows_v[0, i, pl.ds(k * LANES, LANES)] = (
                jnp.zeros((LANES,), jnp.float32))
        return 0
    lax.fori_loop(0, BLK, _z, 0)
    for r in range(ROWS_PER_TILE // BLK):
        pltpu.sync_copy(rows_v.at[0],
                        agg_sp.at[pl.ds(s * ROWS_PER_TILE + r * BLK, BLK)])

    pltpu.sync_copy(pk_hbm.at[w], pk_v)
    pltpu.async_copy(e4_hbm.at[w, 0], e_v.at[0], sem_c.at[0])
    plsc.subcore_barrier()

    def _unpack_src(j, slot):
        for g in range(BLK // LANES):
            pv = pk_v[pl.ds(j * BLK + g * LANES, LANES)]
            sidx_v[slot, pl.ds(g * LANES, LANES)] = pv & 0xFFFF

    _unpack_src(0, 0)
    _unpack_src(1, 1)
    pltpu.async_copy(h_hbm.at[sidx_v.at[0]], rows_v.at[0], sem_g.at[0])
    pltpu.async_copy(h_hbm.at[sidx_v.at[1]], rows_v.at[1], sem_g.at[1])

    def _iter(j, _):
        b = j % NRING
        jl = j % CPB
        cc = j // CPB
        cp = cc % 2

        @pl.when(jnp.logical_and(j < NBLK, jl == 0))
        def _stage():
            pltpu.make_async_copy(e4_hbm.at[w, cc], e_v.at[cp],
                                  sem_c.at[cp]).wait()

            @pl.when(cc + 1 < NCHUNK)
            def _next():
                pltpu.async_copy(e4_hbm.at[w, cc + 1], e_v.at[1 - cp],
                                 sem_c.at[1 - cp])

        @pl.when(j >= 2)
        def _drain():
            b2 = (j - 2) % NRING
            zvec = lax.broadcast(0, (LANES,))
            for q in range(BLK // LANES):
                pltpu.make_async_copy(
                    rows_v.at[b2, pl.ds(q * LANES, LANES)],
                    agg_sp.at[zvec], sem_s.at[b2]).wait()

        @pl.when(j < NBLK)
        def _work():
            pltpu.make_async_copy(h_hbm.at[sidx_v.at[j % 2]],
                                  rows_v.at[b], sem_g.at[b]).wait()

            @pl.when(j + 2 < NBLK)
            def _pref():
                _unpack_src(j + 2, j % 2)
                b1 = (j + 2) % NRING
                pltpu.async_copy(h_hbm.at[sidx_v.at[j % 2]],
                                 rows_v.at[b1], sem_g.at[b1])

            base = jl * BLK
            for g in range(BLK // LANES):
                e16 = e_v[cp, pl.ds(base + g * LANES, LANES)]
                for u in range(LANES):
                    ii = g * LANES + u
                    ee = e16[u]
                    for k in range(D // LANES):
                        fl = pl.ds(k * LANES, LANES)
                        rows_v[b, ii, fl] = rows_v[b, ii, fl] * ee

            for q in range(BLK // LANES):
                pv = pk_v[pl.ds(j * BLK + q * LANES, LANES)]
                ivec = lax.shift_right_logical(pv, 16)
                pltpu.async_copy(rows_v.at[b, pl.ds(q * LANES, LANES)],
                                 agg_sp.at[ivec], sem_s.at[b], add=True)
        return 0
    lax.fori_loop(0, NBLK + 2, _iter, 0)

    plsc.subcore_barrier()
    pltpu.sync_copy(agg_sp.at[pl.ds(s * ROWS_PER_TILE, ROWS_PER_TILE)],
                    out_hbm.at[c, pl.ds(s * ROWS_PER_TILE, ROWS_PER_TILE)])


_SC_MESH = plsc.VectorSubcoreMesh(core_axis_name="c", subcore_axis_name="s",
                                  num_cores=NC, num_subcores=NS)
_SC_PARAMS = pltpu.CompilerParams(needs_layout_passes=False)

_deg_kernel = pl.kernel(
    _deg_body,
    out_type=jax.ShapeDtypeStruct((NC, NPAD), jnp.float32),
    mesh=_SC_MESH,
    compiler_params=_SC_PARAMS,
    scratch_types=[
        pltpu.VMEM((ROWS_PER_TILE,), jnp.float32),
        pltpu.VMEM((BLK,), jnp.float32),
        pltpu.VMEM((NBLK, BLK), jnp.int32),
        pltpu.VMEM_SHARED((NPAD,), jnp.float32),
    ],
)

_gate_kernel = pl.kernel(
    _gate_body,
    out_type=jax.ShapeDtypeStruct((NC * NS, EPT), jnp.float32),
    mesh=_SC_MESH,
    compiler_params=_SC_PARAMS,
    scratch_types=[
        pltpu.VMEM((NPAD,), jnp.float32),
        pltpu.VMEM((NPAD,), jnp.float32),
        pltpu.VMEM((NPAD,), jnp.float32),
        pltpu.VMEM((EPT,), jnp.int32),
        pltpu.VMEM((EPT,), jnp.int32),
        pltpu.VMEM((EPT,), jnp.float32),
        pltpu.SemaphoreType.DMA,
    ],
)

_fa_kernel = pl.kernel(
    _fa_body,
    out_type=jax.ShapeDtypeStruct((NC, NPAD, D), jnp.float32),
    mesh=_SC_MESH,
    compiler_params=_SC_PARAMS,
    scratch_types=[
        pltpu.VMEM((EPT,), jnp.int32),
        pltpu.VMEM((2, BLK), jnp.int32),
        pltpu.VMEM((2, ECH), jnp.float32),
        pltpu.VMEM((NRING, BLK, D), jnp.float32),
        pltpu.SemaphoreType.DMA((NRING,)),
        pltpu.SemaphoreType.DMA((NRING,)),
        pltpu.SemaphoreType.DMA((2,)),
        pltpu.VMEM_SHARED((NPAD, D), jnp.float32),
    ],
)


def _tc_call(body, grid, in_specs, out_specs, out_shape):
    return pl.pallas_call(body, grid=grid, in_specs=in_specs,
                          out_specs=out_specs, out_shape=out_shape)


def kernel(x, edge_index, W1, b1, Wg0, bg0, Wg1, bg1, W2, b2):
    src = edge_index[0]
    dst = edge_index[1]
    pad_e = EPAD - E
    pad_idx = N + (jnp.arange(pad_e, dtype=jnp.int32) % (NPAD - N))
    src3 = jnp.concatenate([src, pad_idx]).reshape(NC * NS, NBLK, BLK)
    dst3 = jnp.concatenate([dst, pad_idx]).reshape(NC * NS, NBLK, BLK)
    srct = src3.reshape(NC * NS, EPT)
    dstt = dst3.reshape(NC * NS, EPT)
    pk = jnp.bitwise_or(srct, jnp.left_shift(dstt, 16))
    x_pad = jnp.zeros((NPAD, D), jnp.float32).at[:N].set(x)

    w1t = W1.T
    w2t = W2.T
    g0 = jnp.zeros((D, D), jnp.float32)
    g0 = g0.at[:, 0].set(Wg0[0, :D]).at[:, 1].set(Wg0[0, D:])
    g1 = jnp.zeros((D, D), jnp.float32)
    g1 = g1.at[:, 0].set(Wg1[0, :D]).at[:, 1].set(Wg1[0, D:])
    bgv0 = jnp.zeros((1, D), jnp.float32).at[0, 0].set(bg0[0])
    bgv1 = jnp.zeros((1, D), jnp.float32).at[0, 0].set(bg1[0])
    b1r = b1.reshape(1, D)
    b2r = b2.reshape(1, D)

    deg2 = _deg_kernel(dst3).reshape(NC, NPAD // D, D)

    RB = 2048
    GP = NPAD // RB
    SB = RB // D
    full = lambda shape: pl.BlockSpec(shape, lambda i: (0,) * len(shape))
    rows = pl.BlockSpec((RB, D), lambda i: (i, 0))

    h_pad, qr0, nrm2 = _tc_call(
        _prep_body, (GP,),
        [rows, full((D, D)), full((1, D)), full((D, D)), full((1, D)),
         pl.BlockSpec((NC, SB, D), lambda i: (0, i, 0))],
        [rows, rows, pl.BlockSpec((SB, D), lambda i: (i, 0))],
        [jax.ShapeDtypeStruct((NPAD, D), jnp.float32),
         jax.ShapeDtypeStruct((NPAD, D), jnp.float32),
         jax.ShapeDtypeStruct((NPAD // D, D), jnp.float32)],
    )(x_pad, w1t, b1r, g0, bgv0, deg2)

    nrm = nrm2.reshape(NPAD)
    e0 = _gate_kernel(qr0[:, 0], qr0[:, 1], nrm, srct, dstt)
    agg0 = _fa_kernel(h_pad, pk, e0.reshape(NC * NS, NCHUNK, ECH))

    h1_pad, qr1 = _tc_call(
        _mid_body, (GP,),
        [rows, pl.BlockSpec((NC, RB, D), lambda i: (0, i, 0)),
         full((D, D)), full((1, D))],
        [rows, rows],
        [jax.ShapeDtypeStruct((NPAD, D), jnp.float32),
         jax.ShapeDtypeStruct((NPAD, D), jnp.float32)],
    )(h_pad, agg0, g1, bgv1)

    e1 = _gate_kernel(qr1[:, 0], qr1[:, 1], nrm, srct, dstt)
    agg1 = _fa_kernel(h1_pad, pk, e1.reshape(NC * NS, NCHUNK, ECH))

    RB2 = 2000
    out = _tc_call(
        _final_body, (N // RB2,),
        [pl.BlockSpec((RB2, D), lambda i: (i, 0)),
         pl.BlockSpec((NC, RB2, D), lambda i: (0, i, 0)),
         full((D, D)), full((1, D))],
        pl.BlockSpec((RB2, D), lambda i: (i, 0)),
        jax.ShapeDtypeStruct((N, D), jnp.float32),
    )(h_pad, agg1, w2t, b2r)

    return out
